# Initial kernel scaffold; baseline (speedup 1.0000x reference)
import jax
import jax.numpy as jnp
from jax.experimental import pallas as pl


def kernel(entity, symbol_emb):
    raise NotImplementedError("write your pallas kernel here")


# speedup vs baseline: 1.3276x; 1.3276x over previous
import jax
import jax.numpy as jnp
from jax import lax
from jax.experimental import pallas as pl
from jax.experimental.pallas import tpu as pltpu
from jax.experimental.pallas import tpu_sc as plsc

_B = 4096
_D = 128
_NC = 2
_NS = 16
_NW = _NC * _NS
_ROWS = 2 * _B
_RPW = _ROWS // _NW
_CHUNK = 128
_NCHUNK = _RPW // _CHUNK


def _body(idx_hbm, table_hbm, left_hbm, right_hbm, idx_v, rows_v, sem):
    wid = lax.axis_index("s") * _NC + lax.axis_index("c")
    pltpu.sync_copy(idx_hbm.at[wid], idx_v)
    copies = [
        pltpu.async_copy(
            table_hbm.at[idx_v.at[c]],
            rows_v.at[pl.ds(c * _CHUNK, _CHUNK)],
            sem,
        )
        for c in range(_NCHUNK)
    ]
    for cp in copies:
        cp.wait()

    half = _NW // 2

    @pl.when(wid < half)
    def _():
        pltpu.sync_copy(rows_v, left_hbm.at[pl.ds(wid * _RPW, _RPW)])

    @pl.when(wid >= half)
    def _():
        pltpu.sync_copy(rows_v, right_hbm.at[pl.ds((wid - half) * _RPW, _RPW)])


_gather = pl.kernel(
    _body,
    out_type=(
        jax.ShapeDtypeStruct((_B, _D), jnp.float32),
        jax.ShapeDtypeStruct((_B, _D), jnp.float32),
    ),
    mesh=plsc.VectorSubcoreMesh(core_axis_name="c", subcore_axis_name="s"),
    scratch_types=[
        pltpu.VMEM((_NCHUNK, _CHUNK), jnp.int32),
        pltpu.VMEM((_RPW, _D), jnp.float32),
        pltpu.SemaphoreType.DMA,
    ],
)


def kernel(entity, symbol_emb):
    idx = entity.astype(jnp.int32).T.reshape(_NW, _NCHUNK, _CHUNK)
    return _gather(idx, symbol_emb)
